# baseline (speedup 1.0000x reference)
import functools

import jax
import jax.numpy as jnp
from jax import lax
from jax.experimental import pallas as pl
from jax.experimental.pallas import tpu as pltpu
from jax.experimental.pallas import tpu_sc as plsc

N = 10000
E = 320000
IN = 128
HID = 256
OUT = 128

BLK = 128
R = 10112
NW = 32
CHUNK = 128
GSZ = 8
NGRP = 10
NCHUNK = GSZ * NGRP
EW = NCHUNK * CHUNK
EPAD = NW * EW
RPT = R // 16
BROWS = 80
TAIL = 72
DW = 16


def _make_sc_agg():
    mesh = plsc.VectorSubcoreMesh(core_axis_name="c", subcore_axis_name="s")

    @functools.partial(
        pl.kernel,
        mesh=mesh,
        out_type=jax.ShapeDtypeStruct((2, R, IN), jnp.float32),
        scratch_types=[
            pltpu.VMEM((GSZ, CHUNK), jnp.int32),
            pltpu.VMEM((GSZ, CHUNK), jnp.int32),
            pltpu.VMEM((CHUNK, IN), jnp.float32),
            pltpu.VMEM((CHUNK, IN), jnp.float32),
            pltpu.VMEM_SHARED((R, IN), jnp.float32),
            pltpu.SemaphoreType.DMA,
            pltpu.SemaphoreType.DMA,
            pltpu.SemaphoreType.DMA,
            pltpu.SemaphoreType.DMA,
        ],
    )
    def body(rows_hbm, srcg_hbm, dstg_hbm, zeros_hbm, out_hbm,
             src_blk, dst_blk, rows0, rows1, acc,
             semg0, semg1, sems0, sems1):
        c = lax.axis_index("c")
        s = lax.axis_index("s")
        wid = c * 16 + s
        gbase = wid * NGRP
        bounce = rows0.at[pl.ds(0, BROWS)]
        pltpu.sync_copy(zeros_hbm, bounce)

        def zbody(k, carry):
            pltpu.sync_copy(bounce, acc.at[pl.ds(s * RPT + k * BROWS, BROWS)])
            return carry

        lax.fori_loop(0, 7, zbody, 0)
        pltpu.sync_copy(bounce.at[pl.ds(0, TAIL)],
                        acc.at[pl.ds(s * RPT + 7 * BROWS, TAIL)])
        plsc.subcore_barrier()

        def drain(rv, sem):
            pltpu.make_async_copy(rows_hbm.at[pl.ds(0, CHUNK)], rv, sem).wait()

        def gbody(g, carry):
            @pl.when(g > 0)
            def _():
                drain(rows0, sems0)
                drain(rows1, sems1)

            pltpu.sync_copy(srcg_hbm.at[gbase + g], src_blk)
            pltpu.sync_copy(dstg_hbm.at[gbase + g], dst_blk)

            def pbody(t, carry2):
                @pl.when(t > 0)
                def _():
                    drain(rows0, sems0)
                pltpu.async_copy(
                    rows_hbm.at[src_blk.at[2 * t]], rows0, semg0).wait()
                pltpu.async_copy(rows0, acc.at[dst_blk.at[2 * t]], sems0,
                                 add=True)

                @pl.when(t > 0)
                def _():
                    drain(rows1, sems1)
                pltpu.async_copy(
                    rows_hbm.at[src_blk.at[2 * t + 1]], rows1, semg1).wait()
                pltpu.async_copy(rows1, acc.at[dst_blk.at[2 * t + 1]], sems1,
                                 add=True)
                return carry2

            lax.fori_loop(0, GSZ // 2, pbody, 0)
            return carry

        lax.fori_loop(0, NGRP, gbody, 0)
        drain(rows0, sems0)
        drain(rows1, sems1)
        plsc.subcore_barrier()

        def obody(k, carry):
            r0 = s * RPT + k * BROWS
            pltpu.sync_copy(acc.at[pl.ds(r0, BROWS)], bounce)
            pltpu.sync_copy(bounce, out_hbm.at[c, pl.ds(r0, BROWS)])
            return carry

        lax.fori_loop(0, 7, obody, 0)
        r7 = s * RPT + 7 * BROWS
        pltpu.sync_copy(acc.at[pl.ds(r7, TAIL)], bounce.at[pl.ds(0, TAIL)])
        pltpu.sync_copy(bounce.at[pl.ds(0, TAIL)],
                        out_hbm.at[c, pl.ds(r7, TAIL)])

    return body


def _make_sc_deg():
    mesh = plsc.VectorSubcoreMesh(core_axis_name="c", subcore_axis_name="s")

    @functools.partial(
        pl.kernel,
        mesh=mesh,
        out_type=jax.ShapeDtypeStruct((2, R, DW), jnp.float32),
        scratch_types=[
            pltpu.VMEM((GSZ, CHUNK), jnp.int32),
            pltpu.VMEM((CHUNK, DW), jnp.float32),
            pltpu.VMEM_SHARED((R, DW), jnp.float32),
        ],
    )
    def body(dstg_hbm, ones16_hbm, deg_hbm, dst_blk, ones_v, acc_deg):
        c = lax.axis_index("c")
        s = lax.axis_index("s")
        wid = c * 16 + s
        gbase = wid * NGRP
        bounce = ones_v.at[pl.ds(0, BROWS)]
        pltpu.sync_copy(ones16_hbm.at[pl.ds(CHUNK, BROWS)], bounce)

        def zbody(k, carry):
            pltpu.sync_copy(bounce,
                            acc_deg.at[pl.ds(s * RPT + k * BROWS, BROWS)])
            return carry

        lax.fori_loop(0, 7, zbody, 0)
        pltpu.sync_copy(bounce.at[pl.ds(0, TAIL)],
                        acc_deg.at[pl.ds(s * RPT + 7 * BROWS, TAIL)])
        pltpu.sync_copy(ones16_hbm.at[pl.ds(0, CHUNK)], ones_v)
        plsc.subcore_barrier()

        def gbody(g, carry):
            pltpu.sync_copy(dstg_hbm.at[gbase + g], dst_blk)

            def sbody(k, carry2):
                pltpu.sync_copy(ones_v, acc_deg.at[dst_blk.at[k]], add=True)
                return carry2

            lax.fori_loop(0, GSZ, sbody, 0)
            return carry

        lax.fori_loop(0, NGRP, gbody, 0)
        plsc.subcore_barrier()

        def obody(k, carry):
            r0 = s * RPT + k * BROWS
            pltpu.sync_copy(acc_deg.at[pl.ds(r0, BROWS)], bounce)
            pltpu.sync_copy(bounce, deg_hbm.at[c, pl.ds(r0, BROWS)])
            return carry

        lax.fori_loop(0, 7, obody, 0)
        r7 = s * RPT + 7 * BROWS
        pltpu.sync_copy(acc_deg.at[pl.ds(r7, TAIL)], bounce.at[pl.ds(0, TAIL)])
        pltpu.sync_copy(bounce.at[pl.ds(0, TAIL)],
                        deg_hbm.at[c, pl.ds(r7, TAIL)])

    return body


_sc_agg = _make_sc_agg()
_sc_deg = _make_sc_deg()


def _dot_t(a, b):
    return lax.dot_general(a, b, (((1,), (1,)), ((), ())),
                           preferred_element_type=jnp.float32)


def _tc1_body(p0, p1, pd0, pd1, xb, w1l, b1l, w1r, w2l,
              h_out, p2_out, dinv_out):
    agg = p0[...] + p1[...]
    deg = pd0[:, :1] + pd1[:, :1]
    dinv = 1.0 / jnp.maximum(deg, 1.0)
    mean = agg * dinv
    h = _dot_t(mean, w1l[...]) + b1l[...] + _dot_t(xb[...], w1r[...])
    h = jnp.maximum(h, 0.0)
    h_out[...] = h
    p2_out[...] = _dot_t(h, w2l[...])
    dinv_out[...] = jnp.broadcast_to(dinv, (BLK, IN))


_tc1 = pl.pallas_call(
    _tc1_body,
    grid=(R // BLK,),
    in_specs=[
        pl.BlockSpec((BLK, IN), lambda i: (i, 0)),
        pl.BlockSpec((BLK, IN), lambda i: (i, 0)),
        pl.BlockSpec((BLK, DW), lambda i: (i, 0)),
        pl.BlockSpec((BLK, DW), lambda i: (i, 0)),
        pl.BlockSpec((BLK, IN), lambda i: (i, 0)),
        pl.BlockSpec((HID, IN), lambda i: (0, 0)),
        pl.BlockSpec((1, HID), lambda i: (0, 0)),
        pl.BlockSpec((HID, IN), lambda i: (0, 0)),
        pl.BlockSpec((OUT, HID), lambda i: (0, 0)),
    ],
    out_specs=[
        pl.BlockSpec((BLK, HID), lambda i: (i, 0)),
        pl.BlockSpec((BLK, OUT), lambda i: (i, 0)),
        pl.BlockSpec((BLK, IN), lambda i: (i, 0)),
    ],
    out_shape=[
        jax.ShapeDtypeStruct((R, HID), jnp.float32),
        jax.ShapeDtypeStruct((R, OUT), jnp.float32),
        jax.ShapeDtypeStruct((R, IN), jnp.float32),
    ],
)


def _tc2_body(q0, q1, dinvf, hb, w2r, b2l, o_out):
    mean2 = (q0[...] + q1[...]) * dinvf[...]
    o_out[...] = mean2 + b2l[...] + _dot_t(hb[...], w2r[...])


_tc2 = pl.pallas_call(
    _tc2_body,
    grid=(R // BLK,),
    in_specs=[
        pl.BlockSpec((BLK, OUT), lambda i: (i, 0)),
        pl.BlockSpec((BLK, OUT), lambda i: (i, 0)),
        pl.BlockSpec((BLK, IN), lambda i: (i, 0)),
        pl.BlockSpec((BLK, HID), lambda i: (i, 0)),
        pl.BlockSpec((OUT, HID), lambda i: (0, 0)),
        pl.BlockSpec((1, OUT), lambda i: (0, 0)),
    ],
    out_specs=pl.BlockSpec((BLK, OUT), lambda i: (i, 0)),
    out_shape=jax.ShapeDtypeStruct((R, OUT), jnp.float32),
)


def kernel(x, edge, W1l, b1l, W1r, W2l, b2l, W2r):
    src = edge[0].astype(jnp.int32)
    dst = edge[1].astype(jnp.int32)
    srcg = jnp.concatenate(
        [src, jnp.zeros((EPAD - E,), jnp.int32)]).reshape(NW * NGRP, GSZ, CHUNK)
    dstg = jnp.concatenate(
        [dst, jnp.full((EPAD - E,), N, jnp.int32)]).reshape(NW * NGRP, GSZ, CHUNK)

    xp = jnp.pad(x, ((0, R - N), (0, 0)))
    zeros = jnp.zeros((BROWS, IN), jnp.float32)
    ones16 = jnp.concatenate([jnp.ones((CHUNK, DW), jnp.float32),
                              jnp.zeros((BROWS, DW), jnp.float32)])

    P = _sc_agg(xp, srcg, dstg, zeros)
    PD = _sc_deg(dstg, ones16)
    h, p2, dinvf = _tc1(P[0], P[1], PD[0], PD[1], xp,
                        W1l, b1l.reshape(1, HID), W1r, W2l)
    Q = _sc_agg(p2, srcg, dstg, zeros)
    out = _tc2(Q[0], Q[1], dinvf, h, W2r, b2l.reshape(1, OUT))
    return out[:N]
